# baseline (speedup 1.0000x reference)
import functools

import jax
import jax.numpy as jnp
from jax import lax
from jax.experimental import pallas as pl
from jax.experimental.pallas import tpu as pltpu
from jax.experimental.pallas import tpu_sc as plsc

B, T, D, K = 16, 576, 64, 8192
N = B * T
NC, NS, L = 2, 16, 16
NW = NC * NS
BPW = N // NW
GROUPS = BPW // L
CHUNK = 96
NCHUNK = BPW // CHUNK

_mesh = plsc.VectorSubcoreMesh(core_axis_name="c", subcore_axis_name="s")


@functools.partial(
    pl.kernel,
    mesh=_mesh,
    out_type=(
        jax.ShapeDtypeStruct((N, D), jnp.float32),
        jax.ShapeDtypeStruct((N,), jnp.float32),
    ),
    scratch_types=[
        pltpu.VMEM((BPW,), jnp.int32),
        pltpu.VMEM((BPW, D), jnp.float32),
        pltpu.VMEM((BPW, D), jnp.float32),
        pltpu.VMEM((BPW,), jnp.float32),
        pltpu.SemaphoreType.DMA,
        pltpu.SemaphoreType.DMA,
    ],
    compiler_params=pltpu.CompilerParams(
        use_tc_tiling_on_sc=False, needs_layout_passes=False
    ),
)
def _sc_vq(idx_hbm, lat_hbm, table_hbm, q_hbm, loss_hbm,
           idx_v, rows_v, lat_v, loss_v, gsem, wsem):
    wid = lax.axis_index("s") * NC + lax.axis_index("c")
    base = wid * BPW
    pltpu.sync_copy(idx_hbm.at[pl.ds(base, BPW)], idx_v)
    copies = [
        pltpu.async_copy(
            table_hbm.at[idx_v.at[pl.ds(c * CHUNK, CHUNK)]],
            rows_v.at[pl.ds(c * CHUNK, CHUNK)],
            gsem,
        )
        for c in range(NCHUNK)
    ]
    pltpu.sync_copy(lat_hbm.at[pl.ds(base, BPW)], lat_v)
    for cp in copies:
        cp.wait()
    wcp = pltpu.async_copy(rows_v, q_hbm.at[pl.ds(base, BPW)], wsem)

    lane = lax.iota(jnp.int32, L)

    def group_body(g, carry):
        rows = lane + g * L
        acc = jnp.zeros((L,), jnp.float32)
        for j in range(D):
            col = jnp.full((L,), j, dtype=jnp.int32)
            qv = plsc.load_gather(rows_v, [rows, col])
            lv = plsc.load_gather(lat_v, [rows, col])
            dv = qv - lv
            acc = acc + dv * dv
        loss_v[pl.ds(g * L, L)] = acc * (1.25 / D)
        return carry

    lax.fori_loop(0, GROUPS, group_body, 0)
    wcp.wait()
    pltpu.sync_copy(loss_v, loss_hbm.at[pl.ds(base, BPW)])


def kernel(gold_encoding_inds, latents, epc, codebook):
    idx = gold_encoding_inds.reshape(N)
    lat_flat = latents.reshape(N, D)
    q_flat, loss = _sc_vq(idx, lat_flat, codebook)
    return (
        q_flat.reshape(B, T, D),
        loss.reshape(B, T),
        gold_encoding_inds.T,
    )


# speedup vs baseline: 1.1685x; 1.1685x over previous
import functools

import jax
import jax.numpy as jnp
from jax import lax
from jax.experimental import pallas as pl
from jax.experimental.pallas import tpu as pltpu
from jax.experimental.pallas import tpu_sc as plsc

B, T, D, K = 16, 576, 64, 8192
N = B * T
NC, NS, L = 2, 16, 16
NW = NC * NS
BPW = N // NW
CHUNK = 96
NCHUNK = BPW // CHUNK
HALF = N // 2

_mesh = plsc.VectorSubcoreMesh(core_axis_name="c", subcore_axis_name="s")


@functools.partial(
    pl.kernel,
    mesh=_mesh,
    out_type=jax.ShapeDtypeStruct((HALF, 2 * D), jnp.float32),
    scratch_types=[
        pltpu.VMEM((BPW,), jnp.int32),
        pltpu.VMEM((BPW, D), jnp.float32),
        pltpu.SemaphoreType.DMA,
    ],
    compiler_params=pltpu.CompilerParams(
        use_tc_tiling_on_sc=False, needs_layout_passes=False
    ),
)
def _sc_gather(idx_hbm, table_hbm, q2_hbm, idx_v, rows_v, sem):
    wid = lax.axis_index("s") * NC + lax.axis_index("c")
    base = wid * BPW
    pltpu.sync_copy(idx_hbm.at[pl.ds(base, BPW)], idx_v)
    copies = [
        pltpu.async_copy(
            table_hbm.at[idx_v.at[pl.ds(c * CHUNK, CHUNK)]],
            rows_v.at[pl.ds(c * CHUNK, CHUNK)],
            sem,
        )
        for c in range(NCHUNK)
    ]
    for cp in copies:
        cp.wait()
    row0 = base - (base // HALF) * HALF

    @pl.when(base < HALF)
    def _():
        pltpu.sync_copy(rows_v, q2_hbm.at[pl.ds(row0, BPW), pl.ds(0, D)])

    @pl.when(base >= HALF)
    def _():
        pltpu.sync_copy(rows_v, q2_hbm.at[pl.ds(row0, BPW), pl.ds(D, D)])


_GBLK = 512
_NBLK = HALF // _GBLK


def _tc_body(q2_ref, l_ref, q_ref, loss_ref):
    h = pl.program_id(1)
    qh = jnp.where(h == 0, q2_ref[:, :D], q2_ref[:, D:])
    d = qh - l_ref[...]
    q_ref[...] = qh
    loss_ref[...] = (jnp.sum(d * d, axis=1) * (1.25 / D))[None, None, :]


def _tc_loss_unpack(q2, lat_flat):
    return pl.pallas_call(
        _tc_body,
        grid=(_NBLK, 2),
        in_specs=[
            pl.BlockSpec((_GBLK, 2 * D), lambda i, h: (i, 0)),
            pl.BlockSpec((_GBLK, D), lambda i, h: (h * _NBLK + i, 0)),
        ],
        out_specs=[
            pl.BlockSpec((_GBLK, D), lambda i, h: (h * _NBLK + i, 0)),
            pl.BlockSpec((1, 1, _GBLK), lambda i, h: (h * _NBLK + i, 0, 0)),
        ],
        out_shape=[
            jax.ShapeDtypeStruct((N, D), jnp.float32),
            jax.ShapeDtypeStruct((2 * _NBLK, 1, _GBLK), jnp.float32),
        ],
    )(q2, lat_flat)


def kernel(gold_encoding_inds, latents, epc, codebook):
    idx = gold_encoding_inds.reshape(N)
    lat_flat = latents.reshape(N, D)
    q2 = _sc_gather(idx, codebook)
    q_flat, loss2 = _tc_loss_unpack(q2, lat_flat)
    return (
        q_flat.reshape(B, T, D),
        loss2.reshape(B, T),
        gold_encoding_inds.T,
    )


# speedup vs baseline: 1.5016x; 1.2851x over previous
import functools

import jax
import jax.numpy as jnp
from jax import lax
from jax.experimental import pallas as pl
from jax.experimental.pallas import tpu as pltpu
from jax.experimental.pallas import tpu_sc as plsc

B, T, D, K = 16, 576, 64, 8192
N = B * T
NC, NS, L = 2, 16, 16
NW = NC * NS
BPW = N // NW
CHUNK = 96
NCHUNK = BPW // CHUNK
HALF = N // 2

_mesh = plsc.VectorSubcoreMesh(core_axis_name="c", subcore_axis_name="s")


@functools.partial(
    pl.kernel,
    mesh=_mesh,
    out_type=jax.ShapeDtypeStruct((HALF, 2 * D), jnp.float32),
    scratch_types=[
        pltpu.VMEM((BPW,), jnp.int32),
        pltpu.VMEM((BPW, D), jnp.float32),
        pltpu.SemaphoreType.DMA,
    ],
    compiler_params=pltpu.CompilerParams(
        use_tc_tiling_on_sc=False, needs_layout_passes=False
    ),
)
def _sc_gather(idx_hbm, table_hbm, q2_hbm, idx_v, rows_v, sem):
    wid = lax.axis_index("s") * NC + lax.axis_index("c")
    base = wid * BPW
    pltpu.sync_copy(idx_hbm.at[pl.ds(base, BPW)], idx_v)
    copies = [
        pltpu.async_copy(
            table_hbm.at[idx_v.at[pl.ds(c * CHUNK, CHUNK)]],
            rows_v.at[pl.ds(c * CHUNK, CHUNK)],
            sem,
        )
        for c in range(NCHUNK)
    ]
    for cp in copies:
        cp.wait()
    row0 = base - (base // HALF) * HALF

    @pl.when(base < HALF)
    def _():
        pltpu.sync_copy(rows_v, q2_hbm.at[pl.ds(row0, BPW), pl.ds(0, D)])

    @pl.when(base >= HALF)
    def _():
        pltpu.sync_copy(rows_v, q2_hbm.at[pl.ds(row0, BPW), pl.ds(D, D)])


def _tc_body(q2_ref, latT_ref, qT_ref, loss_ref):
    h = pl.program_id(1)
    q2t = jnp.transpose(q2_ref[...], (1, 0))
    qT = jnp.where(h == 0, q2t[:D, :], q2t[D:, :])
    d = qT - latT_ref[0]
    qT_ref[0] = qT
    loss_ref[0, 0, :] = jnp.sum(d * d, axis=0) * (1.25 / D)


def _tc_loss_unpack(q2, latT):
    return pl.pallas_call(
        _tc_body,
        grid=(B // 2, 2),
        in_specs=[
            pl.BlockSpec((T, 2 * D), lambda i, h: (i, 0)),
            pl.BlockSpec((1, D, T), lambda i, h: (h * (B // 2) + i, 0, 0)),
        ],
        out_specs=[
            pl.BlockSpec((1, D, T), lambda i, h: (h * (B // 2) + i, 0, 0)),
            pl.BlockSpec((1, 1, T), lambda i, h: (h * (B // 2) + i, 0, 0)),
        ],
        out_shape=[
            jax.ShapeDtypeStruct((B, D, T), jnp.float32),
            jax.ShapeDtypeStruct((B, 1, T), jnp.float32),
        ],
    )(q2, latT)


def kernel(gold_encoding_inds, latents, epc, codebook):
    idx = gold_encoding_inds.reshape(N)
    latT = latents.transpose(0, 2, 1)
    q2 = _sc_gather(idx, codebook)
    qT, loss3 = _tc_loss_unpack(q2, latT)
    return (
        qT.transpose(0, 2, 1),
        loss3.reshape(B, T),
        gold_encoding_inds.T,
    )


# speedup vs baseline: 1.6951x; 1.1289x over previous
import functools

import jax
import jax.numpy as jnp
from jax import lax
from jax.experimental import pallas as pl
from jax.experimental.pallas import tpu as pltpu
from jax.experimental.pallas import tpu_sc as plsc

B, T, D, K = 16, 576, 64, 8192
N = B * T
NC, NS, L = 2, 16, 16
NW = NC * NS
BPW = N // NW
CHUNK = 96
NCHUNK = BPW // CHUNK
HALF = N // 2

_mesh = plsc.VectorSubcoreMesh(core_axis_name="c", subcore_axis_name="s")


@functools.partial(
    pl.kernel,
    mesh=_mesh,
    out_type=jax.ShapeDtypeStruct((HALF, 2 * D), jnp.float32),
    scratch_types=[
        pltpu.VMEM((BPW,), jnp.int32),
        pltpu.VMEM((BPW, D), jnp.float32),
        pltpu.SemaphoreType.DMA,
    ],
    compiler_params=pltpu.CompilerParams(
        use_tc_tiling_on_sc=False, needs_layout_passes=False
    ),
)
def _sc_gather(idx_hbm, table_hbm, q2_hbm, idx_v, rows_v, sem):
    wid = lax.axis_index("s") * NC + lax.axis_index("c")
    base = wid * BPW
    pltpu.sync_copy(idx_hbm.at[pl.ds(base, BPW)], idx_v)
    copies = [
        pltpu.async_copy(
            table_hbm.at[idx_v.at[pl.ds(c * CHUNK, CHUNK)]],
            rows_v.at[pl.ds(c * CHUNK, CHUNK)],
            sem,
        )
        for c in range(NCHUNK)
    ]
    for cp in copies:
        cp.wait()
    row0 = base - (base // HALF) * HALF

    @pl.when(base < HALF)
    def _():
        pltpu.sync_copy(rows_v, q2_hbm.at[pl.ds(row0, BPW), pl.ds(0, D)])

    @pl.when(base >= HALF)
    def _():
        pltpu.sync_copy(rows_v, q2_hbm.at[pl.ds(row0, BPW), pl.ds(D, D)])


_HB = B // 2


def _tc_body(q2_ref, latT_ref, qT_ref, loss_ref):
    q2t = jnp.transpose(q2_ref[...], (1, 0))
    qa, qb = q2t[:D, :], q2t[D:, :]
    da = qa - latT_ref[0, 0]
    db = qb - latT_ref[1, 0]
    qT_ref[0, 0] = qa
    qT_ref[1, 0] = qb
    loss_ref[0, 0, 0, :] = jnp.sum(da * da, axis=0) * (1.25 / D)
    loss_ref[1, 0, 0, :] = jnp.sum(db * db, axis=0) * (1.25 / D)


def _tc_loss_unpack(q2, latT4):
    return pl.pallas_call(
        _tc_body,
        grid=(_HB,),
        in_specs=[
            pl.BlockSpec((T, 2 * D), lambda i: (i, 0)),
            pl.BlockSpec((2, 1, D, T), lambda i: (0, i, 0, 0)),
        ],
        out_specs=[
            pl.BlockSpec((2, 1, D, T), lambda i: (0, i, 0, 0)),
            pl.BlockSpec((2, 1, 1, T), lambda i: (0, i, 0, 0)),
        ],
        out_shape=[
            jax.ShapeDtypeStruct((2, _HB, D, T), jnp.float32),
            jax.ShapeDtypeStruct((2, _HB, 1, T), jnp.float32),
        ],
    )(q2, latT4)


def kernel(gold_encoding_inds, latents, epc, codebook):
    idx = gold_encoding_inds.reshape(N)
    latT4 = latents.transpose(0, 2, 1).reshape(2, _HB, D, T)
    q2 = _sc_gather(idx, codebook)
    qT4, loss4 = _tc_loss_unpack(q2, latT4)
    return (
        qT4.reshape(B, D, T).transpose(0, 2, 1),
        loss4.reshape(B, T),
        gold_encoding_inds.T,
    )
